# baseline (speedup 1.0000x reference)
import functools

import jax
import jax.numpy as jnp
from jax import lax
from jax.experimental import pallas as pl
from jax.experimental.pallas import tpu as pltpu
from jax.experimental.pallas import tpu_sc as plsc

N = 10000
D = 128
E = 320000

NC = 2
NS = 16
NW = NC * NS
EW = 64
RPW = 160
ROWS_PAD = NW * RPW
CH = 8
NSEC = 5
SECR = RPW // NSEC
NB = 4
NPAD = 10112
SEG = NPAD // NS

_MESH = plsc.VectorSubcoreMesh(core_axis_name="c", subcore_axis_name="s")


def _spmm_body(h_hbm, src_hbm, dst_hbm, zeros_hbm, out_hbm,
               src_v0, src_v1, dst_v0, dst_v1, buf0, buf1, acc,
               semi, sg0, sg1):
    c = lax.axis_index("c")
    s = lax.axis_index("s")
    wid = c * NS + s
    base = s * SEG
    row0 = wid * RPW
    srcs = [src_v0, src_v1]
    dsts = [dst_v0, dst_v1]
    ci = pltpu.async_copy(src_hbm.at[pl.ds(row0, SECR)], src_v0, semi)
    cd = pltpu.async_copy(dst_hbm.at[pl.ds(row0, SECR)], dst_v0, semi)
    pltpu.sync_copy(zeros_hbm.at[pl.ds(base, SEG)], acc.at[pl.ds(base, SEG)])
    ci.wait()
    cd.wait()
    plsc.subcore_barrier()

    def make_body(src_v, dst_v):
        def step(j, buf_a, sg_a, buf_b, sg_b):
            pltpu.make_async_copy(h_hbm.at[src_v.at[j]], buf_a, sg_a).wait()

            @pl.when(j < SECR - 1)
            def _():
                pltpu.async_copy(h_hbm.at[src_v.at[j + 1]], buf_b, sg_b)

            pltpu.sync_copy(buf_a, acc.at[dst_v.at[j]], add=True)

        def body(j, carry):
            even = (j & 1) == 0

            @pl.when(even)
            def _():
                step(j, buf0, sg0, buf1, sg1)

            @pl.when(jnp.logical_not(even))
            def _():
                step(j, buf1, sg1, buf0, sg0)

            return carry

        return body

    for sec in range(NSEC):
        bk = sec % 2
        src_v, dst_v = srcs[bk], dsts[bk]
        if sec + 1 < NSEC:
            r_next = row0 + (sec + 1) * SECR
            ci = pltpu.async_copy(src_hbm.at[pl.ds(r_next, SECR)],
                                  srcs[1 - bk], semi)
            cd = pltpu.async_copy(dst_hbm.at[pl.ds(r_next, SECR)],
                                  dsts[1 - bk], semi)
        pltpu.async_copy(h_hbm.at[src_v.at[0]], buf0, sg0)
        lax.fori_loop(0, SECR, make_body(src_v, dst_v), 0)
        if sec + 1 < NSEC:
            ci.wait()
            cd.wait()

    plsc.subcore_barrier()
    pltpu.sync_copy(acc.at[pl.ds(base, SEG)], out_hbm.at[c, pl.ds(base, SEG)])


def _spmm_sc(h, src_rows, dst_rows, zeros):
    kfn = pl.kernel(
        _spmm_body,
        out_type=jax.ShapeDtypeStruct((NC, NPAD, D), jnp.float32),
        mesh=_MESH,
        scratch_types=[
            pltpu.VMEM((SECR, EW), jnp.int32),
            pltpu.VMEM((SECR, EW), jnp.int32),
            pltpu.VMEM((SECR, EW), jnp.int32),
            pltpu.VMEM((SECR, EW), jnp.int32),
            pltpu.VMEM((EW, D), jnp.float32),
            pltpu.VMEM((EW, D), jnp.float32),
            pltpu.VMEM_SHARED((NPAD, D), jnp.float32),
        ] + [pltpu.SemaphoreType.DMA] * 3,
    )
    return kfn(h, src_rows, dst_rows, zeros)


def _deg_body(dst_hbm, zeros_hbm, out_hbm, dst_v, ones_v, acc):
    c = lax.axis_index("c")
    s = lax.axis_index("s")
    wid = c * NS + s
    for i in range(EW // 16):
        ones_v[pl.ds(i * 16, 16)] = jnp.ones((16,), jnp.float32)

    @pl.when(s == 0)
    def _():
        pltpu.sync_copy(zeros_hbm, acc)

    plsc.subcore_barrier()
    row0 = wid * RPW

    def chunk(k, carry):
        r = row0 + k * CH
        pltpu.sync_copy(dst_hbm.at[pl.ds(r, CH)], dst_v)
        for j in range(CH):
            pltpu.sync_copy(ones_v, acc.at[dst_v.at[j]], add=True)
        return carry

    lax.fori_loop(0, RPW // CH, chunk, 0)
    plsc.subcore_barrier()

    @pl.when(s == 0)
    def _():
        pltpu.sync_copy(acc, out_hbm.at[c])


def _deg_sc(dst_rows, zeros_n):
    kfn = pl.kernel(
        _deg_body,
        out_type=jax.ShapeDtypeStruct((NC, NPAD), jnp.float32),
        mesh=_MESH,
        scratch_types=[
            pltpu.VMEM((CH, EW), jnp.int32),
            pltpu.VMEM((EW,), jnp.float32),
            pltpu.VMEM_SHARED((NPAD,), jnp.float32),
        ],
    )
    return kfn(dst_rows, zeros_n)


_BLK = 1000
_GRID = N // _BLK

_row_spec = pl.BlockSpec((_BLK, D), lambda i: (i, 0))
_dis_spec = pl.BlockSpec((_BLK, 1), lambda i: (i, 0))
_w_spec = pl.BlockSpec((D, D), lambda i: (0, 0))
_b_spec = pl.BlockSpec((1, D), lambda i: (0, 0))
_deg_spec = pl.BlockSpec((_BLK, 2), lambda i: (i, 0))
_p0_spec = pl.BlockSpec((1, _BLK, D), lambda i: (0, i, 0))
_p1_spec = pl.BlockSpec((1, _BLK, D), lambda i: (1, i, 0))


def _mm_body(x_ref, w_ref, out_ref):
    out_ref[...] = jnp.dot(x_ref[...], w_ref[...],
                           preferred_element_type=jnp.float32)


def _tc_mm(x, W1):
    return pl.pallas_call(
        _mm_body,
        grid=(_GRID,),
        in_specs=[_row_spec, _w_spec],
        out_specs=_row_spec,
        out_shape=jax.ShapeDtypeStruct((N, D), jnp.float32),
    )(x, W1)


def _scale_body(t_ref, degT_ref, hp_ref, dis_ref):
    cnt = degT_ref[:, 0:1] + degT_ref[:, 1:2] + 1.0
    dis = lax.rsqrt(cnt)
    hp_ref[...] = t_ref[...] * dis
    dis_ref[...] = dis


def _tc_scale(t, degT):
    return pl.pallas_call(
        _scale_body,
        grid=(_GRID,),
        in_specs=[_row_spec, _deg_spec],
        out_specs=[_row_spec, _dis_spec],
        out_shape=[
            jax.ShapeDtypeStruct((N, D), jnp.float32),
            jax.ShapeDtypeStruct((N, 1), jnp.float32),
        ],
    )(t, degT)


def _mid_body(p0_ref, p1_ref, hp_ref, dis_ref, b_ref, w_ref, out_ref):
    dis = dis_ref[...]
    y = dis * (p0_ref[0] + p1_ref[0] + hp_ref[...]) + b_ref[...]
    z = jnp.maximum(y, 0.0)
    out_ref[...] = dis * jnp.dot(z, w_ref[...],
                                 preferred_element_type=jnp.float32)


def _tc_mid(p, hp, dis, b, W):
    return pl.pallas_call(
        _mid_body,
        grid=(_GRID,),
        in_specs=[_p0_spec, _p1_spec, _row_spec, _dis_spec, _b_spec, _w_spec],
        out_specs=_row_spec,
        out_shape=jax.ShapeDtypeStruct((N, D), jnp.float32),
    )(p, p, hp, dis, b, W)


def _post_body(p0_ref, p1_ref, hp_ref, dis_ref, b_ref, x_ref, out_ref):
    y = dis_ref[...] * (p0_ref[0] + p1_ref[0] + hp_ref[...]) + b_ref[...]
    out_ref[...] = y + x_ref[...]


def _tc_post(p, hp, dis, b, x):
    return pl.pallas_call(
        _post_body,
        grid=(_GRID,),
        in_specs=[_p0_spec, _p1_spec, _row_spec, _dis_spec, _b_spec,
                  _row_spec],
        out_specs=_row_spec,
        out_shape=jax.ShapeDtypeStruct((N, D), jnp.float32),
    )(p, p, hp, dis, b, x)


def kernel(x, edge_index, W1, b1, W2, b2, W3, b3):
    src = edge_index[0]
    dst = edge_index[1]
    npad_e = ROWS_PAD * EW - E
    pad_i = jnp.arange(npad_e, dtype=jnp.int32)
    pad_src = (pad_i * 37) % N
    pad_dst = N + (pad_i % (NPAD - N))
    src_rows = jnp.concatenate([src, pad_src]).reshape(ROWS_PAD, EW)
    dst_rows = jnp.concatenate([dst, pad_dst]).reshape(ROWS_PAD, EW)

    zeros = jnp.zeros((NPAD, D), jnp.float32)
    zeros_n = jnp.zeros((NPAD,), jnp.float32)

    degp = _deg_sc(dst_rows, zeros_n)
    t1 = _tc_mm(x, W1)
    degT = jnp.transpose(degp[:, :N])

    b1r = b1.reshape(1, D)
    b2r = b2.reshape(1, D)
    b3r = b3.reshape(1, D)

    hp1, dis = _tc_scale(t1, degT)
    p = _spmm_sc(hp1, src_rows, dst_rows, zeros)
    hp2 = _tc_mid(p, hp1, dis, b1r, W2)
    p = _spmm_sc(hp2, src_rows, dst_rows, zeros)
    hp3 = _tc_mid(p, hp2, dis, b2r, W3)
    p = _spmm_sc(hp3, src_rows, dst_rows, zeros)
    out = _tc_post(p, hp3, dis, b3r, x)
    return (out, x)


# speedup vs baseline: 1.2961x; 1.2961x over previous
import functools

import jax
import jax.numpy as jnp
from jax import lax
from jax.experimental import pallas as pl
from jax.experimental.pallas import tpu as pltpu
from jax.experimental.pallas import tpu_sc as plsc

N = 10000
D = 128
E = 320000

NC = 2
NS = 16
NW = NC * NS
EW = 128
RPW = 80
ROWS_PAD = NW * RPW
CH = 8
NSEC = 5
SECR = RPW // NSEC
NPAD = 10112
SEG = NPAD // NS

_MESH = plsc.VectorSubcoreMesh(core_axis_name="c", subcore_axis_name="s")


def _spmm_body(h_hbm, src_hbm, dst_hbm, zeros_hbm, out_hbm,
               src_v0, src_v1, dst_v0, dst_v1, buf0, buf1, acc,
               semi, sg0, sg1):
    c = lax.axis_index("c")
    s = lax.axis_index("s")
    wid = c * NS + s
    base = s * SEG
    row0 = wid * RPW
    srcs = [src_v0, src_v1]
    dsts = [dst_v0, dst_v1]
    ci = pltpu.async_copy(src_hbm.at[pl.ds(row0, SECR)], src_v0, semi)
    cd = pltpu.async_copy(dst_hbm.at[pl.ds(row0, SECR)], dst_v0, semi)
    pltpu.sync_copy(zeros_hbm.at[pl.ds(base, SEG)], acc.at[pl.ds(base, SEG)])
    ci.wait()
    cd.wait()
    plsc.subcore_barrier()

    def make_body(src_v, dst_v):
        def step(j, buf_a, sg_a, buf_b, sg_b):
            pltpu.make_async_copy(h_hbm.at[src_v.at[j]], buf_a, sg_a).wait()

            @pl.when(j < SECR - 1)
            def _():
                pltpu.async_copy(h_hbm.at[src_v.at[j + 1]], buf_b, sg_b)

            pltpu.sync_copy(buf_a, acc.at[dst_v.at[j]], add=True)

        def body(j, carry):
            even = (j & 1) == 0

            @pl.when(even)
            def _():
                step(j, buf0, sg0, buf1, sg1)

            @pl.when(jnp.logical_not(even))
            def _():
                step(j, buf1, sg1, buf0, sg0)

            return carry

        return body

    for sec in range(NSEC):
        bk = sec % 2
        src_v, dst_v = srcs[bk], dsts[bk]
        if sec + 1 < NSEC:
            r_next = row0 + (sec + 1) * SECR
            ci = pltpu.async_copy(src_hbm.at[pl.ds(r_next, SECR)],
                                  srcs[1 - bk], semi)
            cd = pltpu.async_copy(dst_hbm.at[pl.ds(r_next, SECR)],
                                  dsts[1 - bk], semi)
        pltpu.async_copy(h_hbm.at[src_v.at[0]], buf0, sg0)
        lax.fori_loop(0, SECR, make_body(src_v, dst_v), 0)
        if sec + 1 < NSEC:
            ci.wait()
            cd.wait()

    plsc.subcore_barrier()
    pltpu.sync_copy(acc.at[pl.ds(base, SEG)], out_hbm.at[c, pl.ds(base, SEG)])


def _spmm_sc(h, src_rows, dst_rows, zeros):
    kfn = pl.kernel(
        _spmm_body,
        out_type=jax.ShapeDtypeStruct((NC, NPAD, D), jnp.float32),
        mesh=_MESH,
        scratch_types=[
            pltpu.VMEM((SECR, EW), jnp.int32),
            pltpu.VMEM((SECR, EW), jnp.int32),
            pltpu.VMEM((SECR, EW), jnp.int32),
            pltpu.VMEM((SECR, EW), jnp.int32),
            pltpu.VMEM((EW, D), jnp.float32),
            pltpu.VMEM((EW, D), jnp.float32),
            pltpu.VMEM_SHARED((NPAD, D), jnp.float32),
        ] + [pltpu.SemaphoreType.DMA] * 3,
    )
    return kfn(h, src_rows, dst_rows, zeros)


def _deg_body(dst_hbm, zeros_hbm, out_hbm, dst_v, ones_v, acc):
    c = lax.axis_index("c")
    s = lax.axis_index("s")
    wid = c * NS + s
    for i in range(EW // 16):
        ones_v[pl.ds(i * 16, 16)] = jnp.ones((16,), jnp.float32)

    @pl.when(s == 0)
    def _():
        pltpu.sync_copy(zeros_hbm, acc)

    plsc.subcore_barrier()
    row0 = wid * RPW

    def chunk(k, carry):
        r = row0 + k * CH
        pltpu.sync_copy(dst_hbm.at[pl.ds(r, CH)], dst_v)
        for j in range(CH):
            pltpu.sync_copy(ones_v, acc.at[dst_v.at[j]], add=True)
        return carry

    lax.fori_loop(0, RPW // CH, chunk, 0)
    plsc.subcore_barrier()

    @pl.when(s == 0)
    def _():
        pltpu.sync_copy(acc, out_hbm.at[c])


def _deg_sc(dst_rows, zeros_n):
    kfn = pl.kernel(
        _deg_body,
        out_type=jax.ShapeDtypeStruct((NC, NPAD), jnp.float32),
        mesh=_MESH,
        scratch_types=[
            pltpu.VMEM((CH, EW), jnp.int32),
            pltpu.VMEM((EW,), jnp.float32),
            pltpu.VMEM_SHARED((NPAD,), jnp.float32),
        ],
    )
    return kfn(dst_rows, zeros_n)


_BLK = 1000
_GRID = N // _BLK

_row_spec = pl.BlockSpec((_BLK, D), lambda i: (i, 0))
_dis_spec = pl.BlockSpec((_BLK, 1), lambda i: (i, 0))
_w_spec = pl.BlockSpec((D, D), lambda i: (0, 0))
_b_spec = pl.BlockSpec((1, D), lambda i: (0, 0))
_deg_spec = pl.BlockSpec((_BLK, 2), lambda i: (i, 0))
_p0_spec = pl.BlockSpec((1, _BLK, D), lambda i: (0, i, 0))
_p1_spec = pl.BlockSpec((1, _BLK, D), lambda i: (1, i, 0))


def _mm_body(x_ref, w_ref, out_ref):
    out_ref[...] = jnp.dot(x_ref[...], w_ref[...],
                           preferred_element_type=jnp.float32)


def _tc_mm(x, W1):
    return pl.pallas_call(
        _mm_body,
        grid=(_GRID,),
        in_specs=[_row_spec, _w_spec],
        out_specs=_row_spec,
        out_shape=jax.ShapeDtypeStruct((N, D), jnp.float32),
    )(x, W1)


def _scale_body(t_ref, degT_ref, hp_ref, dis_ref):
    cnt = degT_ref[:, 0:1] + degT_ref[:, 1:2] + 1.0
    dis = lax.rsqrt(cnt)
    hp_ref[...] = t_ref[...] * dis
    dis_ref[...] = dis


def _tc_scale(t, degT):
    return pl.pallas_call(
        _scale_body,
        grid=(_GRID,),
        in_specs=[_row_spec, _deg_spec],
        out_specs=[_row_spec, _dis_spec],
        out_shape=[
            jax.ShapeDtypeStruct((N, D), jnp.float32),
            jax.ShapeDtypeStruct((N, 1), jnp.float32),
        ],
    )(t, degT)


def _mid_body(p0_ref, p1_ref, hp_ref, dis_ref, b_ref, w_ref, out_ref):
    dis = dis_ref[...]
    y = dis * (p0_ref[0] + p1_ref[0] + hp_ref[...]) + b_ref[...]
    z = jnp.maximum(y, 0.0)
    out_ref[...] = dis * jnp.dot(z, w_ref[...],
                                 preferred_element_type=jnp.float32)


def _tc_mid(p, hp, dis, b, W):
    return pl.pallas_call(
        _mid_body,
        grid=(_GRID,),
        in_specs=[_p0_spec, _p1_spec, _row_spec, _dis_spec, _b_spec, _w_spec],
        out_specs=_row_spec,
        out_shape=jax.ShapeDtypeStruct((N, D), jnp.float32),
    )(p, p, hp, dis, b, W)


def _post_body(p0_ref, p1_ref, hp_ref, dis_ref, b_ref, x_ref, out_ref):
    y = dis_ref[...] * (p0_ref[0] + p1_ref[0] + hp_ref[...]) + b_ref[...]
    out_ref[...] = y + x_ref[...]


def _tc_post(p, hp, dis, b, x):
    return pl.pallas_call(
        _post_body,
        grid=(_GRID,),
        in_specs=[_p0_spec, _p1_spec, _row_spec, _dis_spec, _b_spec,
                  _row_spec],
        out_specs=_row_spec,
        out_shape=jax.ShapeDtypeStruct((N, D), jnp.float32),
    )(p, p, hp, dis, b, x)


def kernel(x, edge_index, W1, b1, W2, b2, W3, b3):
    src = edge_index[0]
    dst = edge_index[1]
    npad_e = ROWS_PAD * EW - E
    pad_i = jnp.arange(npad_e, dtype=jnp.int32)
    pad_src = (pad_i * 37) % N
    pad_dst = N + (pad_i % (NPAD - N))
    src_rows = jnp.concatenate([src, pad_src]).reshape(ROWS_PAD, EW)
    dst_rows = jnp.concatenate([dst, pad_dst]).reshape(ROWS_PAD, EW)

    zeros = jnp.zeros((NPAD, D), jnp.float32)
    zeros_n = jnp.zeros((NPAD,), jnp.float32)

    degp = _deg_sc(dst_rows, zeros_n)
    t1 = _tc_mm(x, W1)
    degT = jnp.transpose(degp[:, :N])

    b1r = b1.reshape(1, D)
    b2r = b2.reshape(1, D)
    b3r = b3.reshape(1, D)

    hp1, dis = _tc_scale(t1, degT)
    p = _spmm_sc(hp1, src_rows, dst_rows, zeros)
    hp2 = _tc_mid(p, hp1, dis, b1r, W2)
    p = _spmm_sc(hp2, src_rows, dst_rows, zeros)
    hp3 = _tc_mid(p, hp2, dis, b2r, W3)
    p = _spmm_sc(hp3, src_rows, dst_rows, zeros)
    out = _tc_post(p, hp3, dis, b3r, x)
    return (out, x)
